# baseline (speedup 1.0000x reference)
import jax
import jax.numpy as jnp
from jax import lax
from jax.experimental import pallas as pl
from jax.experimental.pallas import tpu as pltpu
from jax.experimental.pallas import tpu_sc as plsc

N = 16384
D_DRV, D_WK, D_TM = 16, 3, 8
D_OUT = D_DRV + D_WK + D_TM + 1

_info = plsc.get_sparse_core_info()
NC, NS, L = _info.num_cores, _info.num_subcores, _info.num_lanes
NW = NC * NS
B_W = N // NW
CHUNKS = B_W // L


COLS_SP = 16


def _body(drv_idx_hbm, wk_idx_hbm, tm_idx_hbm, dist_hbm,
          wd_t_hbm, wk_t_hbm, wt_t_hbm, out_hbm,
          drv_idx_v, wk_idx_v, tm_idx_v, buf_v,
          wk_tab_v, tm_tab_v, wd_sp, sem, sem2):
    s = lax.axis_index("s")
    c = lax.axis_index("c")
    wid = s * NC + c
    base = wid * B_W

    pltpu.sync_copy(drv_idx_hbm.at[pl.ds(base, B_W)], drv_idx_v)

    @pl.when(s < COLS_SP)
    def _():
        pltpu.sync_copy(wd_t_hbm.at[s], wd_sp.at[s])

    plsc.subcore_barrier()
    copies = [
        pltpu.async_copy(wd_t_hbm.at[col].at[drv_idx_v], buf_v.at[col], sem)
        for col in range(COLS_SP, D_DRV)
    ]
    copies += [
        pltpu.async_copy(wd_sp.at[col].at[drv_idx_v], buf_v.at[col], sem2)
        for col in range(COLS_SP)
    ]

    pltpu.sync_copy(wk_idx_hbm.at[pl.ds(base, B_W)], wk_idx_v)
    pltpu.sync_copy(tm_idx_hbm.at[pl.ds(base, B_W)], tm_idx_v)
    pltpu.sync_copy(dist_hbm.at[pl.ds(base, B_W)], buf_v.at[D_OUT - 1])
    pltpu.sync_copy(wk_t_hbm, wk_tab_v)
    pltpu.sync_copy(wt_t_hbm, tm_tab_v)

    iota = lax.iota(jnp.int32, L)

    def chunk(i, carry):
        r = i * L
        rows = r + iota
        wk16 = plsc.load_gather(wk_idx_v, [rows])
        for col in range(D_WK):
            val = plsc.load_gather(wk_tab_v, [iota * 0 + col, wk16])
            buf_v[D_DRV + col, pl.ds(r, L)] = val
        tm16 = plsc.load_gather(tm_idx_v, [rows])
        for col in range(D_TM):
            val = plsc.load_gather(tm_tab_v, [iota * 0 + col, tm16])
            buf_v[D_DRV + D_WK + col, pl.ds(r, L)] = val
        return carry

    lax.fori_loop(0, CHUNKS, chunk, 0)

    for cp in copies:
        cp.wait()

    pltpu.sync_copy(buf_v, out_hbm.at[:, pl.ds(base, B_W)])


@jax.jit
def _run(drv_idx, wk_idx, tm_idx, dist, wd_t, wk_t, wt_t):
    mesh = plsc.VectorSubcoreMesh(core_axis_name="c", subcore_axis_name="s")
    f = pl.kernel(
        _body, mesh=mesh,
        compiler_params=pltpu.CompilerParams(
            needs_layout_passes=False, use_tc_tiling_on_sc=False),
        out_type=jax.ShapeDtypeStruct((D_OUT, N), jnp.float32),
        scratch_types=[
            pltpu.VMEM((B_W,), jnp.int32),
            pltpu.VMEM((B_W,), jnp.int32),
            pltpu.VMEM((B_W,), jnp.int32),
            pltpu.VMEM((D_OUT, B_W), jnp.float32),
            pltpu.VMEM((D_WK, 7), jnp.float32),
            pltpu.VMEM((D_TM, 1440), jnp.float32),
            pltpu.VMEM_SHARED((COLS_SP, 24000), jnp.float32),
            pltpu.SemaphoreType.DMA,
            pltpu.SemaphoreType.DMA,
        ],
    )
    return f(drv_idx, wk_idx, tm_idx, dist, wd_t, wk_t, wt_t)


def kernel(driverID, weekID, timeID, dist, W_driver, W_week, W_time):
    drv_idx = driverID.astype(jnp.int32).reshape(-1)
    wk_idx = weekID.astype(jnp.int32).reshape(-1)
    tm_idx = timeID.astype(jnp.int32).reshape(-1)
    out_t = _run(drv_idx, wk_idx, tm_idx, dist.reshape(-1),
                 W_driver.T, W_week.T, W_time.T)
    return out_t.T


# speedup vs baseline: 1.0375x; 1.0375x over previous
import jax
import jax.numpy as jnp
from jax import lax
from jax.experimental import pallas as pl
from jax.experimental.pallas import tpu as pltpu
from jax.experimental.pallas import tpu_sc as plsc

N = 16384
D_DRV, D_WK, D_TM = 16, 3, 8
D_OUT = D_DRV + D_WK + D_TM + 1
V_DRV, V_WK, V_TM = 24000, 7, 1440

_info = plsc.get_sparse_core_info()
NC, NS, L = _info.num_cores, _info.num_subcores, _info.num_lanes
NW = NC * NS
B_W = N // NW
CHUNKS = B_W // L

COLS_SP = 13

OFF_WD = 0
OFF_WT = OFF_WD + D_DRV * V_DRV
OFF_WK = OFF_WT + D_TM * V_TM
TAB_LEN = OFF_WK + D_WK * V_WK


def _body(drv_idx_hbm, wk_idx_hbm, tm_idx_hbm, dist_hbm, tab_hbm, out_hbm,
          drv_idx_v, wk_idx_v, tm_idx_v, buf_v,
          wk_tab_v, tm_tab_v, wd_sp, sem, sem2):
    s = lax.axis_index("s")
    c = lax.axis_index("c")
    wid = s * NC + c
    base = wid * B_W

    pltpu.sync_copy(drv_idx_hbm.at[pl.ds(base, B_W)], drv_idx_v)

    @pl.when(s < COLS_SP)
    def _():
        pltpu.sync_copy(tab_hbm.at[pl.ds(s * V_DRV, V_DRV)], wd_sp.at[s])

    pltpu.sync_copy(wk_idx_hbm.at[pl.ds(base, B_W)], wk_idx_v)
    pltpu.sync_copy(tm_idx_hbm.at[pl.ds(base, B_W)], tm_idx_v)
    pltpu.sync_copy(dist_hbm.at[pl.ds(base, B_W)], buf_v.at[D_OUT - 1])
    pltpu.sync_copy(tab_hbm.at[pl.ds(OFF_WK, D_WK * V_WK)], wk_tab_v)
    pltpu.sync_copy(tab_hbm.at[pl.ds(OFF_WT, D_TM * V_TM)], tm_tab_v)

    plsc.subcore_barrier()
    copies = [
        pltpu.async_copy(
            tab_hbm.at[pl.ds(col * V_DRV, V_DRV)].at[drv_idx_v],
            buf_v.at[col], sem)
        for col in range(COLS_SP, D_DRV)
    ]
    copies += [
        pltpu.async_copy(wd_sp.at[col].at[drv_idx_v], buf_v.at[col], sem2)
        for col in range(COLS_SP)
    ]

    iota = lax.iota(jnp.int32, L)

    def chunk(i, carry):
        r = i * L
        rows = r + iota
        wk16 = plsc.load_gather(wk_idx_v, [rows])
        for col in range(D_WK):
            val = plsc.load_gather(wk_tab_v, [wk16 + col * V_WK])
            buf_v[D_DRV + col, pl.ds(r, L)] = val
        tm16 = plsc.load_gather(tm_idx_v, [rows])
        for col in range(D_TM):
            val = plsc.load_gather(tm_tab_v, [tm16 + col * V_TM])
            buf_v[D_DRV + D_WK + col, pl.ds(r, L)] = val
        return carry

    lax.fori_loop(0, CHUNKS, chunk, 0)

    for cp in copies:
        cp.wait()

    pltpu.sync_copy(buf_v, out_hbm.at[:, pl.ds(base, B_W)])


@jax.jit
def _run(drv_idx, wk_idx, tm_idx, dist, tab):
    mesh = plsc.VectorSubcoreMesh(core_axis_name="c", subcore_axis_name="s")
    f = pl.kernel(
        _body, mesh=mesh,
        compiler_params=pltpu.CompilerParams(
            needs_layout_passes=False, use_tc_tiling_on_sc=False),
        out_type=jax.ShapeDtypeStruct((D_OUT, N), jnp.float32),
        scratch_types=[
            pltpu.VMEM((B_W,), jnp.int32),
            pltpu.VMEM((B_W,), jnp.int32),
            pltpu.VMEM((B_W,), jnp.int32),
            pltpu.VMEM((D_OUT, B_W), jnp.float32),
            pltpu.VMEM((D_WK * V_WK,), jnp.float32),
            pltpu.VMEM((D_TM * V_TM,), jnp.float32),
            pltpu.VMEM_SHARED((COLS_SP, V_DRV), jnp.float32),
            pltpu.SemaphoreType.DMA,
            pltpu.SemaphoreType.DMA,
        ],
    )
    return f(drv_idx, wk_idx, tm_idx, dist, tab)


def kernel(driverID, weekID, timeID, dist, W_driver, W_week, W_time):
    drv_idx = driverID.astype(jnp.int32).reshape(-1)
    wk_idx = weekID.astype(jnp.int32).reshape(-1)
    tm_idx = timeID.astype(jnp.int32).reshape(-1)
    tab = jnp.concatenate([
        W_driver.T.reshape(-1), W_time.T.reshape(-1), W_week.T.reshape(-1)])
    out_t = _run(drv_idx, wk_idx, tm_idx, dist.reshape(-1), tab)
    return out_t.T


# speedup vs baseline: 1.0425x; 1.0048x over previous
import jax
import jax.numpy as jnp
from jax import lax
from jax.experimental import pallas as pl
from jax.experimental.pallas import tpu as pltpu
from jax.experimental.pallas import tpu_sc as plsc

N = 16384
D_DRV, D_WK, D_TM = 16, 3, 8
D_OUT = D_DRV + D_WK + D_TM + 1
V_DRV, V_WK, V_TM = 24000, 7, 1440

_info = plsc.get_sparse_core_info()
NC, NS, L = _info.num_cores, _info.num_subcores, _info.num_lanes
NW = NC * NS
B_W = N // NW
CHUNKS = B_W // L

COLS_SP = 13

OFF_WD = 0
OFF_WT = OFF_WD + D_DRV * V_DRV
OFF_WK = OFF_WT + D_TM * V_TM
TAB_LEN = OFF_WK + D_WK * V_WK


def _body(drv_idx_hbm, wk_idx_hbm, tm_idx_hbm, dist_hbm, tab_hbm, out_hbm,
          drv_idx_v, wk_idx_v, tm_idx_v, buf_v,
          wk_tab_v, tm_tab_v, wd_sp, sem, sem2, sem3):
    s = lax.axis_index("s")
    c = lax.axis_index("c")
    wid = s * NC + c
    base = wid * B_W

    pltpu.sync_copy(drv_idx_hbm.at[pl.ds(base, B_W)], drv_idx_v)

    @pl.when(s < COLS_SP)
    def _():
        pltpu.sync_copy(tab_hbm.at[pl.ds(s * V_DRV, V_DRV)], wd_sp.at[s])

    pltpu.sync_copy(wk_idx_hbm.at[pl.ds(base, B_W)], wk_idx_v)
    pltpu.sync_copy(tm_idx_hbm.at[pl.ds(base, B_W)], tm_idx_v)
    pltpu.sync_copy(dist_hbm.at[pl.ds(base, B_W)], buf_v.at[D_OUT - 1])
    pltpu.sync_copy(tab_hbm.at[pl.ds(OFF_WK, D_WK * V_WK)], wk_tab_v)
    pltpu.sync_copy(tab_hbm.at[pl.ds(OFF_WT, D_TM * V_TM)], tm_tab_v)

    plsc.subcore_barrier()
    copies = [
        pltpu.async_copy(
            tab_hbm.at[pl.ds(col * V_DRV, V_DRV)].at[drv_idx_v],
            buf_v.at[col], sem)
        for col in range(COLS_SP, D_DRV)
    ]
    copies += [
        pltpu.async_copy(wd_sp.at[col].at[drv_idx_v], buf_v.at[col], sem2)
        for col in range(COLS_SP)
    ]

    iota = lax.iota(jnp.int32, L)

    def chunk(i, carry):
        r = i * L
        rows = r + iota
        wk16 = plsc.load_gather(wk_idx_v, [rows])
        for col in range(D_WK):
            val = plsc.load_gather(wk_tab_v, [wk16 + col * V_WK])
            buf_v[D_DRV + col, pl.ds(r, L)] = val
        tm16 = plsc.load_gather(tm_idx_v, [rows])
        for col in range(D_TM):
            val = plsc.load_gather(tm_tab_v, [tm16 + col * V_TM])
            buf_v[D_DRV + D_WK + col, pl.ds(r, L)] = val
        return carry

    lax.fori_loop(0, CHUNKS, chunk, 0)

    w1 = pltpu.async_copy(
        buf_v.at[pl.ds(D_DRV, D_OUT - D_DRV)],
        out_hbm.at[pl.ds(D_DRV, D_OUT - D_DRV), pl.ds(base, B_W)], sem3)
    for cp in copies:
        cp.wait()
    w1.wait()
    pltpu.sync_copy(buf_v.at[pl.ds(0, D_DRV)],
                    out_hbm.at[pl.ds(0, D_DRV), pl.ds(base, B_W)])


@jax.jit
def _run(drv_idx, wk_idx, tm_idx, dist, tab):
    mesh = plsc.VectorSubcoreMesh(core_axis_name="c", subcore_axis_name="s")
    f = pl.kernel(
        _body, mesh=mesh,
        compiler_params=pltpu.CompilerParams(
            needs_layout_passes=False, use_tc_tiling_on_sc=False),
        out_type=jax.ShapeDtypeStruct((D_OUT, N), jnp.float32),
        scratch_types=[
            pltpu.VMEM((B_W,), jnp.int32),
            pltpu.VMEM((B_W,), jnp.int32),
            pltpu.VMEM((B_W,), jnp.int32),
            pltpu.VMEM((D_OUT, B_W), jnp.float32),
            pltpu.VMEM((D_WK * V_WK,), jnp.float32),
            pltpu.VMEM((D_TM * V_TM,), jnp.float32),
            pltpu.VMEM_SHARED((COLS_SP, V_DRV), jnp.float32),
            pltpu.SemaphoreType.DMA,
            pltpu.SemaphoreType.DMA,
            pltpu.SemaphoreType.DMA,
        ],
    )
    return f(drv_idx, wk_idx, tm_idx, dist, tab)


def kernel(driverID, weekID, timeID, dist, W_driver, W_week, W_time):
    drv_idx = driverID.astype(jnp.int32).reshape(-1)
    wk_idx = weekID.astype(jnp.int32).reshape(-1)
    tm_idx = timeID.astype(jnp.int32).reshape(-1)
    tab = jnp.concatenate([
        W_driver.T.reshape(-1), W_time.T.reshape(-1), W_week.T.reshape(-1)])
    out_t = _run(drv_idx, wk_idx, tm_idx, dist.reshape(-1), tab)
    return out_t.T


# speedup vs baseline: 1.0453x; 1.0027x over previous
import jax
import jax.numpy as jnp
from jax import lax
from jax.experimental import pallas as pl
from jax.experimental.pallas import tpu as pltpu
from jax.experimental.pallas import tpu_sc as plsc

N = 16384
D_DRV, D_WK, D_TM = 16, 3, 8
D_OUT = D_DRV + D_WK + D_TM + 1
V_DRV, V_WK, V_TM = 24000, 7, 1440

_info = plsc.get_sparse_core_info()
NC, NS, L = _info.num_cores, _info.num_subcores, _info.num_lanes
NW = NC * NS
B_W = N // NW
CHUNKS = B_W // L

COLS_SP = 13

OFF_WD = 0
OFF_WT = OFF_WD + D_DRV * V_DRV
OFF_WK = OFF_WT + D_TM * V_TM
TAB_LEN = OFF_WK + D_WK * V_WK


def _body(drv_idx_hbm, wk_idx_hbm, tm_idx_hbm, dist_hbm, tab_hbm, out_hbm,
          drv_idx_v, wk_idx_v, tm_idx_v, buf_v,
          wk_tab_v, tm_tab_v, wd_sp, sem, sem2, sem3):
    s = lax.axis_index("s")
    c = lax.axis_index("c")
    wid = s * NC + c
    base = wid * B_W

    pltpu.sync_copy(drv_idx_hbm.at[pl.ds(base, B_W)], drv_idx_v)

    @pl.when(s < COLS_SP)
    def _():
        pltpu.sync_copy(tab_hbm.at[pl.ds(s * V_DRV, V_DRV)], wd_sp.at[s])

    pltpu.sync_copy(wk_idx_hbm.at[pl.ds(base, B_W)], wk_idx_v)
    pltpu.sync_copy(tm_idx_hbm.at[pl.ds(base, B_W)], tm_idx_v)
    pltpu.sync_copy(dist_hbm.at[pl.ds(base, B_W)], buf_v.at[D_OUT - 1])
    pltpu.sync_copy(tab_hbm.at[pl.ds(OFF_WK, D_WK * V_WK)], wk_tab_v)
    pltpu.sync_copy(tab_hbm.at[pl.ds(OFF_WT, D_TM * V_TM)], tm_tab_v)

    copies = [
        pltpu.async_copy(
            tab_hbm.at[pl.ds(col * V_DRV, V_DRV)].at[drv_idx_v],
            buf_v.at[col], sem)
        for col in range(COLS_SP, D_DRV)
    ]
    plsc.subcore_barrier()
    copies += [
        pltpu.async_copy(wd_sp.at[col].at[drv_idx_v], buf_v.at[col], sem2)
        for col in range(COLS_SP)
    ]

    iota = lax.iota(jnp.int32, L)

    def chunk(i, carry):
        r = i * L
        rows = r + iota
        wk16 = plsc.load_gather(wk_idx_v, [rows])
        for col in range(D_WK):
            val = plsc.load_gather(wk_tab_v, [wk16 + col * V_WK])
            buf_v[D_DRV + col, pl.ds(r, L)] = val
        tm16 = plsc.load_gather(tm_idx_v, [rows])
        for col in range(D_TM):
            val = plsc.load_gather(tm_tab_v, [tm16 + col * V_TM])
            buf_v[D_DRV + D_WK + col, pl.ds(r, L)] = val
        return carry

    lax.fori_loop(0, CHUNKS, chunk, 0)

    w1 = pltpu.async_copy(
        buf_v.at[pl.ds(D_DRV, D_OUT - D_DRV)],
        out_hbm.at[pl.ds(D_DRV, D_OUT - D_DRV), pl.ds(base, B_W)], sem3)
    for cp in copies:
        cp.wait()
    w1.wait()
    pltpu.sync_copy(buf_v.at[pl.ds(0, D_DRV)],
                    out_hbm.at[pl.ds(0, D_DRV), pl.ds(base, B_W)])


@jax.jit
def _run(drv_idx, wk_idx, tm_idx, dist, tab):
    mesh = plsc.VectorSubcoreMesh(core_axis_name="c", subcore_axis_name="s")
    f = pl.kernel(
        _body, mesh=mesh,
        compiler_params=pltpu.CompilerParams(
            needs_layout_passes=False, use_tc_tiling_on_sc=False),
        out_type=jax.ShapeDtypeStruct((D_OUT, N), jnp.float32),
        scratch_types=[
            pltpu.VMEM((B_W,), jnp.int32),
            pltpu.VMEM((B_W,), jnp.int32),
            pltpu.VMEM((B_W,), jnp.int32),
            pltpu.VMEM((D_OUT, B_W), jnp.float32),
            pltpu.VMEM((D_WK * V_WK,), jnp.float32),
            pltpu.VMEM((D_TM * V_TM,), jnp.float32),
            pltpu.VMEM_SHARED((COLS_SP, V_DRV), jnp.float32),
            pltpu.SemaphoreType.DMA,
            pltpu.SemaphoreType.DMA,
            pltpu.SemaphoreType.DMA,
        ],
    )
    return f(drv_idx, wk_idx, tm_idx, dist, tab)


def kernel(driverID, weekID, timeID, dist, W_driver, W_week, W_time):
    drv_idx = driverID.astype(jnp.int32).reshape(-1)
    wk_idx = weekID.astype(jnp.int32).reshape(-1)
    tm_idx = timeID.astype(jnp.int32).reshape(-1)
    tab = jnp.concatenate([
        W_driver.T.reshape(-1), W_time.T.reshape(-1), W_week.T.reshape(-1)])
    out_t = _run(drv_idx, wk_idx, tm_idx, dist.reshape(-1), tab)
    return out_t.T
